# Initial kernel scaffold; baseline (speedup 1.0000x reference)
import jax
import jax.numpy as jnp
from jax.experimental import pallas as pl


def kernel(x, W, b):
    raise NotImplementedError("write your pallas kernel here")


# speedup vs baseline: 1.1922x; 1.1922x over previous
import jax
import jax.numpy as jnp
from jax.experimental import pallas as pl
from jax.experimental.pallas import tpu as pltpu

D_MODEL = 4096
NUM_EXPERTS = 64
TOP_K = 8
BLOCK = 512


def _gate_kernel(x_ref, w_ref, b_ref, vals_ref, idx_ref):
    x = x_ref[...].astype(jnp.bfloat16)
    w = w_ref[...].astype(jnp.bfloat16)
    logits = jax.lax.dot_general(
        x, w, (((1,), (1,)), ((), ())),
        preferred_element_type=jnp.float32,
    ) + b_ref[...]
    m = jnp.max(logits, axis=-1, keepdims=True)
    e = jnp.exp(logits - m)
    p = e / jnp.sum(e, axis=-1, keepdims=True)

    iota = jax.lax.broadcasted_iota(jnp.int32, p.shape, 1)
    s = p
    vals = []
    idxs = []
    for _ in range(TOP_K):
        mk = jnp.max(s, axis=-1, keepdims=True)
        ik = jnp.min(jnp.where(s == mk, iota, NUM_EXPERTS), axis=-1, keepdims=True)
        vals.append(mk)
        idxs.append(ik)
        s = jnp.where(iota == ik, -1.0, s)
    vals_ref[...] = jnp.concatenate(vals, axis=-1)
    idx_ref[...] = jnp.concatenate(idxs, axis=-1)


@jax.jit
def kernel(x, W, b):
    tokens = x.shape[0]
    b2 = b.reshape(1, NUM_EXPERTS)
    vals, idx = pl.pallas_call(
        _gate_kernel,
        grid=(tokens // BLOCK,),
        in_specs=[
            pl.BlockSpec((BLOCK, D_MODEL), lambda i: (i, 0)),
            pl.BlockSpec((NUM_EXPERTS, D_MODEL), lambda i: (0, 0)),
            pl.BlockSpec((1, NUM_EXPERTS), lambda i: (0, 0)),
        ],
        out_specs=[
            pl.BlockSpec((BLOCK, TOP_K), lambda i: (i, 0)),
            pl.BlockSpec((BLOCK, TOP_K), lambda i: (i, 0)),
        ],
        out_shape=[
            jax.ShapeDtypeStruct((tokens, TOP_K), jnp.float32),
            jax.ShapeDtypeStruct((tokens, TOP_K), jnp.int32),
        ],
        compiler_params=pltpu.CompilerParams(
            dimension_semantics=("parallel",),
        ),
    )(x, W, b2)
    return vals, idx


# speedup vs baseline: 1.4721x; 1.2347x over previous
import jax
import jax.numpy as jnp
from jax.experimental import pallas as pl
from jax.experimental.pallas import tpu as pltpu

D_MODEL = 4096
NUM_EXPERTS = 64
TOP_K = 8
BLOCK = 512


def _gate_kernel(x_ref, w_ref, b_ref, vals_ref, idx_ref):
    x = x_ref[...].astype(jnp.bfloat16)
    w = w_ref[...].astype(jnp.bfloat16)
    logits = jax.lax.dot_general(
        x, w, (((1,), (1,)), ((), ())),
        preferred_element_type=jnp.float32,
    ) + b_ref[...]
    m = jnp.max(logits, axis=-1, keepdims=True)
    e = jnp.exp(logits - m)
    p = e / jnp.sum(e, axis=-1, keepdims=True)

    s = p.T
    iota = jax.lax.broadcasted_iota(jnp.int32, s.shape, 0)
    vals = []
    idxs = []
    for _ in range(TOP_K):
        mk = jnp.max(s, axis=0, keepdims=True)
        ik = jnp.min(jnp.where(s == mk, iota, NUM_EXPERTS), axis=0, keepdims=True)
        vals.append(mk)
        idxs.append(ik)
        s = jnp.where(iota == ik, -1.0, s)
    vals_ref[...] = jnp.concatenate(vals, axis=0).T
    idx_ref[...] = jnp.concatenate(idxs, axis=0).T


@jax.jit
def kernel(x, W, b):
    tokens = x.shape[0]
    b2 = b.reshape(1, NUM_EXPERTS)
    vals, idx = pl.pallas_call(
        _gate_kernel,
        grid=(tokens // BLOCK,),
        in_specs=[
            pl.BlockSpec((BLOCK, D_MODEL), lambda i: (i, 0)),
            pl.BlockSpec((NUM_EXPERTS, D_MODEL), lambda i: (0, 0)),
            pl.BlockSpec((1, NUM_EXPERTS), lambda i: (0, 0)),
        ],
        out_specs=[
            pl.BlockSpec((BLOCK, TOP_K), lambda i: (i, 0)),
            pl.BlockSpec((BLOCK, TOP_K), lambda i: (i, 0)),
        ],
        out_shape=[
            jax.ShapeDtypeStruct((tokens, TOP_K), jnp.float32),
            jax.ShapeDtypeStruct((tokens, TOP_K), jnp.int32),
        ],
        compiler_params=pltpu.CompilerParams(
            dimension_semantics=("parallel",),
        ),
    )(x, W, b2)
    return vals, idx


# speedup vs baseline: 1.6066x; 1.0914x over previous
import jax
import jax.numpy as jnp
from jax.experimental import pallas as pl
from jax.experimental.pallas import tpu as pltpu

D_MODEL = 4096
NUM_EXPERTS = 64
TOP_K = 8
BLOCK = 1024


def _gate_kernel(x_ref, w_ref, b_ref, vals_ref, idx_ref):
    x = x_ref[...].astype(jnp.bfloat16)
    w = w_ref[...].astype(jnp.bfloat16)
    logits = jax.lax.dot_general(
        x, w, (((1,), (1,)), ((), ())),
        preferred_element_type=jnp.float32,
    ) + b_ref[...]
    m = jnp.max(logits, axis=-1, keepdims=True)
    e = jnp.exp(logits - m)
    p = e / jnp.sum(e, axis=-1, keepdims=True)

    s = p.T
    iota = jax.lax.broadcasted_iota(jnp.int32, s.shape, 0)
    vals = []
    idxs = []
    for _ in range(TOP_K):
        mk = jnp.max(s, axis=0, keepdims=True)
        ik = jnp.min(jnp.where(s == mk, iota, NUM_EXPERTS), axis=0, keepdims=True)
        vals.append(mk)
        idxs.append(ik)
        s = jnp.where(iota == ik, -1.0, s)
    vals_ref[...] = jnp.concatenate(vals, axis=0).T
    idx_ref[...] = jnp.concatenate(idxs, axis=0).T


@jax.jit
def kernel(x, W, b):
    tokens = x.shape[0]
    b2 = b.reshape(1, NUM_EXPERTS)
    vals, idx = pl.pallas_call(
        _gate_kernel,
        grid=(tokens // BLOCK,),
        in_specs=[
            pl.BlockSpec((BLOCK, D_MODEL), lambda i: (i, 0)),
            pl.BlockSpec((NUM_EXPERTS, D_MODEL), lambda i: (0, 0)),
            pl.BlockSpec((1, NUM_EXPERTS), lambda i: (0, 0)),
        ],
        out_specs=[
            pl.BlockSpec((BLOCK, TOP_K), lambda i: (i, 0)),
            pl.BlockSpec((BLOCK, TOP_K), lambda i: (i, 0)),
        ],
        out_shape=[
            jax.ShapeDtypeStruct((tokens, TOP_K), jnp.float32),
            jax.ShapeDtypeStruct((tokens, TOP_K), jnp.int32),
        ],
        compiler_params=pltpu.CompilerParams(
            dimension_semantics=("parallel",),
        ),
    )(x, W, b2)
    return vals, idx
